# baseline (device time: 101144 ns/iter reference)
import numpy as np

import jax
import jax.numpy as jnp
from jax import lax
from jax.experimental import pallas as pl
from jax.experimental.pallas import tpu as pltpu

N_DEV = 8
B, SQ, D = 2, 512, 1024
T = B * SQ
HL, DH = 8, 128
CH = T // N_DEV
SCALE = 0.08838834764831843
WIRE = jnp.bfloat16


def _rope_tables():
    inv = 1.0 / (10000.0 ** (np.arange(0, DH, 2) / DH))
    pos = np.arange(SQ)[:, None] * inv[None, :]
    ck, sk = np.cos(pos), np.sin(pos)
    cos2 = np.concatenate([ck, ck], axis=1)
    sin2 = np.concatenate([-sk, sk], axis=1)
    cos_t = np.concatenate([cos2, cos2], axis=0)
    sin_t = np.concatenate([sin2, sin2], axis=0)
    return cos_t.astype(np.float32), sin_t.astype(np.float32)


_COS, _SIN = _rope_tables()


def _deinterleave_cols(w):
    return w.reshape(D, HL, DH // 2, 2).transpose(0, 1, 3, 2).reshape(D, HL * DH)


def _vid(d):
    return d ^ ((d >> 1) & 1)


def _body(x_ref, wq_ref, wk_ref, wv_ref, wo_ref, cos_ref, sin_ref, out_ref,
          qb, kb, vb, fb, sb1, sb2, sb3, rb1, rb2, rb3,
          ab1, ab2, ab3, gr1, gr2, gr3, ssem, rsem):
    me = lax.axis_index("i")
    vm = _vid(me)
    prt = [_vid(vm ^ 1), _vid(vm ^ 2), _vid(vm ^ 4)]

    barrier_sem = pltpu.get_barrier_semaphore()
    for nbr in prt:
        pl.semaphore_signal(barrier_sem, inc=1, device_id=(nbr,),
                            device_id_type=pl.DeviceIdType.MESH)
    pl.semaphore_wait(barrier_sem, 3)

    def exchange(k, src, dst, partner):
        return pltpu.make_async_remote_copy(
            src_ref=src, dst_ref=dst,
            send_sem=ssem.at[k], recv_sem=rsem.at[k],
            device_id=(partner,), device_id_type=pl.DeviceIdType.MESH,
        )

    cos = cos_ref[:, :]
    sin = sin_ref[:, :]
    fb[:, :] = jnp.dot(x_ref[:, :], wq_ref[:, :], preferred_element_type=jnp.float32)
    for h in range(HL):
        cs = slice(h * DH, (h + 1) * DH)
        q = fb[:, cs]
        qb[:, cs] = (q * cos + pltpu.roll(q, 64, 1) * sin).astype(WIRE)
    fb[:, :] = jnp.dot(x_ref[:, :], wk_ref[:, :], preferred_element_type=jnp.float32)
    for h in range(HL):
        cs = slice(h * DH, (h + 1) * DH)
        k = fb[:, cs]
        kb[:, cs] = (k * cos + pltpu.roll(k, 64, 1) * sin).astype(WIRE)
    vb[:, :] = jnp.dot(x_ref[:, :], wv_ref[:, :],
                       preferred_element_type=jnp.float32).astype(WIRE)

    def attn_and_partial(lo):
        rows = pl.ds(lo, SQ)
        for h in range(HL):
            cs = slice(h * DH, (h + 1) * DH)
            q = qb[rows, cs]
            k = kb[rows, cs]
            v = vb[rows, cs]
            s = lax.dot_general(q, k, (((1,), (1,)), ((), ())),
                                preferred_element_type=jnp.float32) * SCALE
            m = jnp.max(s, axis=1, keepdims=True)
            e = jnp.exp(s - m)
            den = jnp.sum(e, axis=1, keepdims=True)
            ctx = jnp.dot(e.astype(WIRE), v, preferred_element_type=jnp.float32) / den
            vb[rows, cs] = ctx.astype(WIRE)
        out_ref[rows, :] = jnp.dot(vb[rows, :], wo_ref[:, :],
                                   preferred_element_type=jnp.float32)

    bit2 = (vm >> 2) & 1
    bit1 = (vm >> 1) & 1
    bit0 = vm & 1
    lo_send1 = (1 - bit2) * 512
    lo_keep1 = bit2 * 512

    attn_and_partial(lo_send1)
    sb1[:, :] = out_ref[pl.ds(lo_send1, 512), :].astype(WIRE)
    ex1 = exchange(0, sb1, rb1, prt[2])
    ex1.start()
    attn_and_partial(lo_keep1)
    ex1.wait()
    out_ref[pl.ds(lo_keep1, 512), :] = (
        out_ref[pl.ds(lo_keep1, 512), :] + rb1[:, :].astype(jnp.float32)
    )

    lo_send2 = lo_keep1 + (1 - bit1) * 256
    lo_keep2 = lo_keep1 + bit1 * 256
    sb2[:, :] = out_ref[pl.ds(lo_send2, 256), :].astype(WIRE)
    ex2 = exchange(1, sb2, rb2, prt[1])
    ex2.start()
    ex2.wait()
    out_ref[pl.ds(lo_keep2, 256), :] = (
        out_ref[pl.ds(lo_keep2, 256), :] + rb2[:, :].astype(jnp.float32)
    )

    lo_send3 = lo_keep2 + (1 - bit0) * 128
    lo_keep3 = lo_keep2 + bit0 * 128
    sb3[:, :] = out_ref[pl.ds(lo_send3, 128), :].astype(WIRE)
    ex3 = exchange(2, sb3, rb3, prt[0])
    ex3.start()
    ex3.wait()
    out_ref[pl.ds(lo_keep3, 128), :] = (
        out_ref[pl.ds(lo_keep3, 128), :] + rb3[:, :].astype(jnp.float32)
    )

    lo = lo_keep3
    ab1[:, :] = out_ref[pl.ds(lo, 128), :].astype(WIRE)
    ex4 = exchange(3, ab1, gr1, prt[0])
    ex4.start()
    ex4.wait()
    out_ref[pl.ds(lo + (1 - 2 * bit0) * 128, 128), :] = gr1[:, :].astype(jnp.float32)
    lo = lo - bit0 * 128

    ab2[:, :] = out_ref[pl.ds(lo, 256), :].astype(WIRE)
    ex5 = exchange(4, ab2, gr2, prt[1])
    ex5.start()
    ex5.wait()
    out_ref[pl.ds(lo + (1 - 2 * bit1) * 256, 256), :] = gr2[:, :].astype(jnp.float32)
    lo = lo - bit1 * 256

    ab3[:, :] = out_ref[pl.ds(lo, 512), :].astype(WIRE)
    ex6 = exchange(5, ab3, gr3, prt[2])
    ex6.start()
    ex6.wait()
    out_ref[pl.ds(lo + (1 - 2 * bit2) * 512, 512), :] = gr3[:, :].astype(jnp.float32)


def kernel(x, Wq, Wk, Wv, Wo):
    x2 = x.reshape(T, D).astype(WIRE)
    wq = _deinterleave_cols(Wq).astype(WIRE)
    wk = _deinterleave_cols(Wk).astype(WIRE)
    wv = Wv.astype(WIRE)
    wo = Wo.astype(WIRE)
    cos_t = jnp.asarray(_COS)
    sin_t = jnp.asarray(_SIN)

    out = pl.pallas_call(
        _body,
        out_shape=jax.ShapeDtypeStruct((T, D), jnp.float32),
        in_specs=[pl.BlockSpec(memory_space=pltpu.VMEM)] * 7,
        out_specs=pl.BlockSpec(memory_space=pltpu.VMEM),
        scratch_shapes=[
            pltpu.VMEM((T, HL * DH), WIRE),
            pltpu.VMEM((T, HL * DH), WIRE),
            pltpu.VMEM((T, HL * DH), WIRE),
            pltpu.VMEM((T, HL * DH), jnp.float32),
            pltpu.VMEM((512, D), WIRE),
            pltpu.VMEM((256, D), WIRE),
            pltpu.VMEM((128, D), WIRE),
            pltpu.VMEM((512, D), WIRE),
            pltpu.VMEM((256, D), WIRE),
            pltpu.VMEM((128, D), WIRE),
            pltpu.VMEM((128, D), WIRE),
            pltpu.VMEM((256, D), WIRE),
            pltpu.VMEM((512, D), WIRE),
            pltpu.VMEM((128, D), WIRE),
            pltpu.VMEM((256, D), WIRE),
            pltpu.VMEM((512, D), WIRE),
            pltpu.SemaphoreType.DMA((6,)),
            pltpu.SemaphoreType.DMA((6,)),
        ],
        compiler_params=pltpu.CompilerParams(
            collective_id=0, vmem_limit_bytes=100 * 1024 * 1024
        ),
    )(x2, wq, wk, wv, wo, cos_t, sin_t)
    return out.reshape(B, SQ, D)


# device time: 80513 ns/iter; 1.2562x vs baseline; 1.2562x over previous
import numpy as np

import jax
import jax.numpy as jnp
from jax import lax
from jax.experimental import pallas as pl
from jax.experimental.pallas import tpu as pltpu

N_DEV = 8
B, SQ, D = 2, 512, 1024
T = B * SQ
HL, DH = 8, 128
SCALE = 0.08838834764831843
WIRE = jnp.bfloat16

COLS = [(0, 384), (384, 768), (768, 1024)]
SBITS = [[2, 1, 0], [1, 0, 2], [0, 2, 1]]
RS_ROWS = [512, 256, 128]
AG_ROWS = [128, 256, 512]


def _rope_tables():
    inv = 1.0 / (10000.0 ** (np.arange(0, DH, 2) / DH))
    pos = np.arange(SQ)[:, None] * inv[None, :]
    ck, sk = np.cos(pos), np.sin(pos)
    cos2 = np.concatenate([ck, ck], axis=1)
    sin2 = np.concatenate([-sk, sk], axis=1)
    cos_t = np.concatenate([cos2, cos2], axis=0)
    sin_t = np.concatenate([sin2, sin2], axis=0)
    return cos_t.astype(np.float32), sin_t.astype(np.float32)


_COS, _SIN = _rope_tables()


def _deinterleave_cols(w):
    return w.reshape(D, HL, DH // 2, 2).transpose(0, 1, 3, 2).reshape(D, HL * DH)


def _vid(d):
    return d ^ ((d >> 1) & 1)


def _body(*refs):
    (x_ref, wq_ref, wk_ref, wv_ref, wo_ref,
     cosq_ref, sinq_ref, cos_ref, sin_ref, out_ref) = refs[:10]
    qb, kb, vb = refs[10:13]
    sbufs = refs[13:31]
    rbufs = refs[31:49]
    ssem, rsem = refs[49], refs[50]

    me = lax.axis_index("i")
    vm = _vid(me)
    bits = [vm & 1, (vm >> 1) & 1, (vm >> 2) & 1]

    barrier_sem = pltpu.get_barrier_semaphore()
    for b in range(3):
        pl.semaphore_signal(barrier_sem, inc=1, device_id=(_vid(vm ^ (1 << b)),),
                            device_id_type=pl.DeviceIdType.MESH)
    pl.semaphore_wait(barrier_sem, 3)

    def exchange(slot, p, bitpos):
        return pltpu.make_async_remote_copy(
            src_ref=sbufs[slot * 3 + p], dst_ref=rbufs[slot * 3 + p],
            send_sem=ssem.at[slot * 3 + p], recv_sem=rsem.at[slot * 3 + p],
            device_id=(_vid(vm ^ (1 << bitpos)),),
            device_id_type=pl.DeviceIdType.MESH,
        )

    qb[:, :] = jnp.dot(x_ref[:, :], wq_ref[:, :], preferred_element_type=jnp.float32)
    kb[:, :] = jnp.dot(x_ref[:, :], wk_ref[:, :], preferred_element_type=jnp.float32)
    vb[:, :] = jnp.dot(x_ref[:, :], wv_ref[:, :], preferred_element_type=jnp.float32)
    for h in range(HL):
        cs = slice(h * DH, (h + 1) * DH)
        q = qb[:, cs]
        qb[:, cs] = q * cosq_ref[:, :] + pltpu.roll(q, 64, 1) * sinq_ref[:, :]
        k = kb[:, cs]
        kb[:, cs] = k * cos_ref[:, :] + pltpu.roll(k, 64, 1) * sin_ref[:, :]

    for b in range(B):
        rows = slice(b * SQ, (b + 1) * SQ)
        for h in range(HL):
            cs = slice(h * DH, (h + 1) * DH)
            s = lax.dot_general(qb[rows, cs], kb[rows, cs],
                                (((1,), (1,)), ((), ())),
                                preferred_element_type=jnp.float32)
            e = jnp.exp(s)
            den = jnp.sum(e, axis=1, keepdims=True)
            vb[rows, cs] = jnp.dot(e, vb[rows, cs],
                                   preferred_element_type=jnp.float32) / den
        out_ref[rows, :] = jnp.dot(vb[rows, :], wo_ref[:, :],
                                   preferred_element_type=jnp.float32)

    los = [0, 0, 0]
    for k in range(3):
        half = RS_ROWS[k]
        exs, keeps = [], []
        for p in range(3):
            c0, c1 = COLS[p]
            bp = bits[SBITS[p][k]]
            lo_send = los[p] + (1 - bp) * half
            lo_keep = los[p] + bp * half
            sbufs[k * 3 + p][:, :] = out_ref[pl.ds(lo_send, half), c0:c1].astype(WIRE)
            ex = exchange(k, p, SBITS[p][k])
            ex.start()
            exs.append(ex)
            keeps.append(lo_keep)
        for p in range(3):
            c0, c1 = COLS[p]
            exs[p].wait()
            out_ref[pl.ds(keeps[p], half), c0:c1] = (
                out_ref[pl.ds(keeps[p], half), c0:c1]
                + rbufs[k * 3 + p][:, :].astype(jnp.float32)
            )
        los = keeps

    for j in range(3):
        ln = AG_ROWS[j]
        slot = 3 + j
        exs, plos, nlos = [], [], []
        for p in range(3):
            c0, c1 = COLS[p]
            bitpos = SBITS[p][2 - j]
            bp = bits[bitpos]
            sbufs[slot * 3 + p][:, :] = out_ref[pl.ds(los[p], ln), c0:c1].astype(WIRE)
            ex = exchange(slot, p, bitpos)
            ex.start()
            exs.append(ex)
            plos.append(los[p] + (1 - 2 * bp) * ln)
            nlos.append(los[p] - bp * ln)
        for p in range(3):
            c0, c1 = COLS[p]
            exs[p].wait()
            out_ref[pl.ds(plos[p], ln), c0:c1] = rbufs[slot * 3 + p][:, :].astype(
                jnp.float32)
        los = nlos


def kernel(x, Wq, Wk, Wv, Wo):
    x2 = x.reshape(T, D)
    wq = _deinterleave_cols(Wq)
    wk = _deinterleave_cols(Wk)
    cos_t = jnp.asarray(_COS)
    sin_t = jnp.asarray(_SIN)
    cosq_t = jnp.asarray(_COS * np.float32(SCALE))
    sinq_t = jnp.asarray(_SIN * np.float32(SCALE))

    comm_shapes = []
    for rows in RS_ROWS + AG_ROWS:
        for (c0, c1) in COLS:
            comm_shapes.append(pltpu.VMEM((rows, c1 - c0), WIRE))

    out = pl.pallas_call(
        _body,
        out_shape=jax.ShapeDtypeStruct((T, D), jnp.float32),
        in_specs=[pl.BlockSpec(memory_space=pltpu.VMEM)] * 9,
        out_specs=pl.BlockSpec(memory_space=pltpu.VMEM),
        scratch_shapes=(
            [pltpu.VMEM((T, HL * DH), jnp.float32)] * 3
            + comm_shapes
            + comm_shapes
            + [pltpu.SemaphoreType.DMA((18,)),
               pltpu.SemaphoreType.DMA((18,))]
        ),
        compiler_params=pltpu.CompilerParams(
            collective_id=0, vmem_limit_bytes=100 * 1024 * 1024
        ),
    )(x2, wq, wk, Wv, Wo, cosq_t, sinq_t, cos_t, sin_t)
    return out.reshape(B, SQ, D)


# device time: 79895 ns/iter; 1.2660x vs baseline; 1.0077x over previous
import numpy as np

import jax
import jax.numpy as jnp
from jax import lax
from jax.experimental import pallas as pl
from jax.experimental.pallas import tpu as pltpu

N_DEV = 8
B, SQ, D = 2, 512, 1024
T = B * SQ
HL, DH = 8, 128
SCALE = 0.08838834764831843
WIRE = jnp.bfloat16

COLS = [(0, 384), (384, 768), (768, 1024)]
SBITS = [[2, 1, 0], [1, 0, 2], [0, 2, 1]]
N_SLOTS = 4


def _rope_tables():
    inv = 1.0 / (10000.0 ** (np.arange(0, DH, 2) / DH))
    pos = np.arange(SQ)[:, None] * inv[None, :]
    ck, sk = np.cos(pos), np.sin(pos)
    cos2 = np.concatenate([ck, ck], axis=1)
    sin2 = np.concatenate([-sk, sk], axis=1)
    cos_t = np.concatenate([cos2, cos2], axis=0)
    sin_t = np.concatenate([sin2, sin2], axis=0)
    return cos_t.astype(np.float32), sin_t.astype(np.float32)


_COS, _SIN = _rope_tables()


def _deinterleave_cols(w):
    return w.reshape(D, HL, DH // 2, 2).transpose(0, 1, 3, 2).reshape(D, HL * DH)


def _vid(d):
    return d ^ ((d >> 1) & 1)


def _body(*refs):
    (x_ref, wq_ref, wk_ref, wv_ref, wo_ref,
     cosq_ref, sinq_ref, cos_ref, sin_ref, out_ref) = refs[:10]
    qb, kb, vb = refs[10:13]
    sbufs = refs[13:25]
    rbufs = refs[25:37]
    ssem, rsem = refs[37], refs[38]

    me = lax.axis_index("i")
    vm = _vid(me)
    bits = [vm & 1, (vm >> 1) & 1, (vm >> 2) & 1]

    barrier_sem = pltpu.get_barrier_semaphore()
    for b in range(3):
        pl.semaphore_signal(barrier_sem, inc=1, device_id=(_vid(vm ^ (1 << b)),),
                            device_id_type=pl.DeviceIdType.MESH)
    pl.semaphore_wait(barrier_sem, 3)

    def exchange(slot, p, bitpos):
        return pltpu.make_async_remote_copy(
            src_ref=sbufs[slot * 3 + p], dst_ref=rbufs[slot * 3 + p],
            send_sem=ssem.at[slot * 3 + p], recv_sem=rsem.at[slot * 3 + p],
            device_id=(_vid(vm ^ (1 << bitpos)),),
            device_id_type=pl.DeviceIdType.MESH,
        )

    qb[:, :] = jnp.dot(x_ref[:, :], wq_ref[:, :], preferred_element_type=jnp.float32)
    kb[:, :] = jnp.dot(x_ref[:, :], wk_ref[:, :], preferred_element_type=jnp.float32)
    vb[:, :] = jnp.dot(x_ref[:, :], wv_ref[:, :], preferred_element_type=jnp.float32)
    for h in range(HL):
        cs = slice(h * DH, (h + 1) * DH)
        q = qb[:, cs]
        qb[:, cs] = q * cosq_ref[:, :] + pltpu.roll(q, 64, 1) * sinq_ref[:, :]
        k = kb[:, cs]
        kb[:, cs] = k * cos_ref[:, :] + pltpu.roll(k, 64, 1) * sin_ref[:, :]

    def compute_batch(b):
        rows = slice(b * SQ, (b + 1) * SQ)
        for h in range(HL):
            cs = slice(h * DH, (h + 1) * DH)
            s = lax.dot_general(qb[rows, cs], kb[rows, cs],
                                (((1,), (1,)), ((), ())),
                                preferred_element_type=jnp.float32)
            e = jnp.exp(s)
            den = jnp.sum(e, axis=1, keepdims=True)
            vb[rows, cs] = jnp.dot(e, vb[rows, cs],
                                   preferred_element_type=jnp.float32) / den
        out_ref[rows, :] = jnp.dot(vb[rows, :], wo_ref[:, :],
                                   preferred_element_type=jnp.float32)

    ex0, lo_sends, lo_keeps = [], [], []
    for p in range(3):
        bp = bits[SBITS[p][0]]
        lo_sends.append((1 - bp) * 512)
        lo_keeps.append(bp * 512)
        ex0.append(exchange(0, p, SBITS[p][0]))

    for b in range(B):
        compute_batch(b)
        for p in range(3):
            c0, c1 = COLS[p]

            @pl.when(lo_sends[p] == b * 512)
            def _(p=p, c0=c0, c1=c1, b=b):
                sbufs[p][:, :] = out_ref[b * 512:(b + 1) * 512, c0:c1].astype(WIRE)
                ex0[p].start()

    for p in range(3):
        c0, c1 = COLS[p]
        ex0[p].wait()
        out_ref[pl.ds(lo_keeps[p], 512), c0:c1] = (
            out_ref[pl.ds(lo_keeps[p], 512), c0:c1]
            + rbufs[p][:, :].astype(jnp.float32)
        )

    for s in (1, 2):
        exs = []
        for p in range(3):
            c0, c1 = COLS[p]
            sbufs[s * 3 + p][:, :] = out_ref[pl.ds(lo_keeps[p], 512),
                                             c0:c1].astype(WIRE)
            ex = exchange(s, p, SBITS[p][s])
            ex.start()
            exs.append(ex)
        for p in range(3):
            c0, c1 = COLS[p]
            exs[p].wait()
            out_ref[pl.ds(lo_keeps[p], 512), c0:c1] = (
                out_ref[pl.ds(lo_keeps[p], 512), c0:c1]
                + rbufs[s * 3 + p][:, :].astype(jnp.float32)
            )

    exs = []
    for p in range(3):
        c0, c1 = COLS[p]
        sbufs[9 + p][:, :] = out_ref[pl.ds(lo_keeps[p], 512), c0:c1].astype(WIRE)
        ex = exchange(3, p, SBITS[p][0])
        ex.start()
        exs.append(ex)
    for p in range(3):
        c0, c1 = COLS[p]
        exs[p].wait()
        out_ref[pl.ds(lo_sends[p], 512), c0:c1] = rbufs[9 + p][:, :].astype(
            jnp.float32)


def kernel(x, Wq, Wk, Wv, Wo):
    x2 = x.reshape(T, D)
    wq = _deinterleave_cols(Wq)
    wk = _deinterleave_cols(Wk)
    cos_t = jnp.asarray(_COS)
    sin_t = jnp.asarray(_SIN)
    cosq_t = jnp.asarray(_COS * np.float32(SCALE))
    sinq_t = jnp.asarray(_SIN * np.float32(SCALE))

    comm_shapes = []
    for _slot in range(N_SLOTS):
        for (c0, c1) in COLS:
            comm_shapes.append(pltpu.VMEM((512, c1 - c0), WIRE))

    out = pl.pallas_call(
        _body,
        out_shape=jax.ShapeDtypeStruct((T, D), jnp.float32),
        in_specs=[pl.BlockSpec(memory_space=pltpu.VMEM)] * 9,
        out_specs=pl.BlockSpec(memory_space=pltpu.VMEM),
        scratch_shapes=(
            [pltpu.VMEM((T, HL * DH), jnp.float32)] * 3
            + comm_shapes
            + comm_shapes
            + [pltpu.SemaphoreType.DMA((12,)),
               pltpu.SemaphoreType.DMA((12,))]
        ),
        compiler_params=pltpu.CompilerParams(
            collective_id=0, vmem_limit_bytes=100 * 1024 * 1024
        ),
    )(x2, wq, wk, Wv, Wo, cosq_t, sinq_t, cos_t, sin_t)
    return out.reshape(B, SQ, D)
